# baseline (speedup 1.0000x reference)
import functools

import jax
import jax.numpy as jnp
from jax import lax
from jax.experimental import pallas as pl

B, C, T = 128, 64, 1024
EMB = 32
TK = 64
POOL = 4
TOPK = 8
HID = 64
HEADS = 4
G1 = HID * HEADS
G2 = HID
NCLS = 2
TP = T // POOL


def _elu(v):
    return jnp.where(v > 0, v, jnp.exp(jnp.minimum(v, 0.0)) - 1.0)


def _lrelu(v):
    return jnp.where(v >= 0, v, 0.2 * v)


def _graph_kernel(x_ref, wc_ref, t1_ref, a2_ref, b2_ref,
                  wl1_ref, bl1_ref, wr1_ref, br1_ref, we1_ref, attbd_ref,
                  gb1_ref, wl2_ref, bl2_ref, wr2_ref, br2_ref, we2_ref,
                  att2_ref, gb2_ref, g_ref):
    f32 = jnp.float32
    xr = x_ref[0]

    xm = xr - jnp.mean(xr, axis=1, keepdims=True)
    var = jnp.sum(xm * xm, axis=1, keepdims=True) * (1.0 / (T - 1))
    xs = xm / (jnp.sqrt(var) + 1e-8)
    corr = lax.dot_general(xs, xs, (((1,), (1,)), ((), ())),
                           preferred_element_type=f32) * (1.0 / (T - 1))
    corr = jnp.clip(corr, -1.0, 1.0)
    ri = lax.broadcasted_iota(jnp.int32, (C, C), 0)
    ci = lax.broadcasted_iota(jnp.int32, (C, C), 1)
    score = jnp.where(ri == ci, -1.0, jnp.abs(corr))
    sA = score[:, :, None]
    sB = score[:, None, :]
    ii = lax.broadcasted_iota(jnp.int32, (C, C, C), 1)
    ip = lax.broadcasted_iota(jnp.int32, (C, C, C), 2)
    better = (sB > sA) | ((sB == sA) & (ip < ii))
    cnt = jnp.sum(better.astype(f32), axis=2)
    maskf = jnp.where(cnt < float(TOPK), 1.0, 0.0)
    ef = (corr * maskf).T
    vmf = maskf.T
    vmf3 = vmf[:, :, None]

    xpad = jnp.concatenate(
        [jnp.zeros((C, TK // 2), f32), xr, jnp.zeros((C, TK // 2), f32)],
        axis=1)
    wc = wc_ref[...]
    acc = jnp.zeros((C, T), f32)
    for k in range(TK):
        acc = acc + wc[:, k:k + 1] * lax.slice(xpad, (0, k), (C, k + T))
    y = _elu(acc + t1_ref[...])
    y4 = y.reshape(C, TP, POOL)
    z = jnp.sum(y4, axis=2) * (1.0 / POOL)
    a2 = a2_ref[...].reshape(1, EMB, 1)
    b2 = b2_ref[...].reshape(1, EMB, 1)
    q = _elu(z[:, None, :] * a2 + b2)
    nf = jnp.sum(q, axis=2) * (1.0 / TP)

    xl = jnp.dot(nf, wl1_ref[...], preferred_element_type=f32) + bl1_ref[...]
    xr1 = jnp.dot(nf, wr1_ref[...], preferred_element_type=f32) + br1_ref[...]
    s3 = (xr1[:, None, :] + xl[None, :, :]
          + ef[:, :, None] * we1_ref[...].reshape(1, 1, G1))
    m3 = _lrelu(s3).reshape(C * C, G1)
    logit = jnp.dot(m3, attbd_ref[...],
                    preferred_element_type=f32).reshape(C, C, HEADS)
    lm = jnp.where(vmf3 > 0.5, logit, -1e30)
    lmax = jnp.max(lm, axis=1, keepdims=True)
    aexp = jnp.where(vmf3 > 0.5, jnp.exp(lm - lmax), 0.0)
    den = jnp.sum(aexp, axis=1, keepdims=True)
    alpha = aexp / (den + 1e-16)
    outs = []
    for h in range(HEADS):
        outs.append(jnp.dot(alpha[:, :, h], xl[:, h * HID:(h + 1) * HID],
                            preferred_element_type=f32))
    h1 = _elu(jnp.concatenate(outs, axis=1) + gb1_ref[...])

    xl2 = jnp.dot(h1, wl2_ref[...], preferred_element_type=f32) + bl2_ref[...]
    xr2 = jnp.dot(h1, wr2_ref[...], preferred_element_type=f32) + br2_ref[...]
    s32 = (xr2[:, None, :] + xl2[None, :, :]
           + ef[:, :, None] * we2_ref[...].reshape(1, 1, G2))
    m32 = _lrelu(s32).reshape(C * C, G2)
    logit2 = jnp.dot(m32, att2_ref[...],
                     preferred_element_type=f32).reshape(C, C)
    lm2 = jnp.where(vmf > 0.5, logit2, -1e30)
    lmax2 = jnp.max(lm2, axis=1, keepdims=True)
    aexp2 = jnp.where(vmf > 0.5, jnp.exp(lm2 - lmax2), 0.0)
    den2 = jnp.sum(aexp2, axis=1, keepdims=True)
    alpha2 = aexp2 / (den2 + 1e-16)
    h2 = _elu(jnp.dot(alpha2, xl2, preferred_element_type=f32) + gb2_ref[...])

    g_ref[0] = jnp.sum(h2, axis=0, keepdims=True) * (1.0 / C)


def _head_kernel(g_ref, fc1w, fc1b, fc2w, fc2b, p1w, p1b, p2w, p2b,
                 al1, al2, c1w, c1b, c2w, c2b, out_ref):
    f32 = jnp.float32
    g = g_ref[...]
    a1 = jnp.tanh(jnp.dot(g, fc1w[...], preferred_element_type=f32) + fc1b[...])
    a2 = jnp.tanh(jnp.dot(g, fc2w[...], preferred_element_type=f32) + fc2b[...])
    gg = (g
          + (jnp.dot(a1, p1w[...], preferred_element_type=f32) + p1b[...]) * al1[...]
          + (jnp.dot(a2 * a2, p2w[...], preferred_element_type=f32) + p2b[...]) * al2[...])
    zz = jnp.maximum(jnp.dot(gg, c1w[...], preferred_element_type=f32) + c1b[...], 0.0)
    out_ref[...] = jnp.dot(zz, c2w[...], preferred_element_type=f32) + c2b[...]


def _bcast(shape):
    nd = len(shape)
    return pl.BlockSpec(shape, lambda *_b, _n=nd: (0,) * _n)


@jax.jit
def kernel(x, params):
    p = params
    f32 = jnp.float32
    s1 = p['bn1_g'] / jnp.sqrt(p['bn1_v'] + 1e-5)
    wc = (p['dw_w'][:, 0, :] * s1[:, None]).astype(f32)
    t1 = (p['bn1_b'] - p['bn1_m'] * s1).reshape(C, 1).astype(f32)
    s2 = p['bn2_g'] / jnp.sqrt(p['bn2_v'] + 1e-5)
    a2 = (p['pw_w'] * s2).reshape(1, EMB).astype(f32)
    b2 = (p['bn2_b'] - p['bn2_m'] * s2).reshape(1, EMB).astype(f32)
    attbd = (p['g1_att'][:, :, None]
             * jnp.eye(HEADS, dtype=f32)[:, None, :]).reshape(G1, HEADS)
    att2 = p['g2_att'].reshape(G2, 1)

    operands = [
        x, wc, t1, a2, b2,
        p['g1_wl'], p['g1_bl'].reshape(1, G1),
        p['g1_wr'], p['g1_br'].reshape(1, G1),
        p['g1_we'].reshape(1, G1), attbd, p['g1_bias'].reshape(1, G1),
        p['g2_wl'], p['g2_bl'].reshape(1, G2),
        p['g2_wr'], p['g2_br'].reshape(1, G2),
        p['g2_we'].reshape(1, G2), att2, p['g2_bias'].reshape(1, G2),
    ]
    in_specs = [pl.BlockSpec((1, C, T), lambda b: (b, 0, 0))]
    in_specs += [_bcast(op.shape) for op in operands[1:]]
    g = pl.pallas_call(
        _graph_kernel,
        grid=(B,),
        in_specs=in_specs,
        out_specs=pl.BlockSpec((1, 1, G2), lambda b: (b, 0, 0)),
        out_shape=jax.ShapeDtypeStruct((B, 1, G2), f32),
    )(*operands)
    g = g.reshape(B, G2)

    head_ops = [
        g, p['fc1_w'], p['fc1_b'].reshape(1, G2),
        p['fc2_w'], p['fc2_b'].reshape(1, G2),
        p['p1_w'], p['p1_b'].reshape(1, G2),
        p['p2_w'], p['p2_b'].reshape(1, G2),
        p['alpha1'].reshape(1, G2), p['alpha2'].reshape(1, G2),
        p['c1_w'], p['c1_b'].reshape(1, G2),
        p['c2_w'], p['c2_b'].reshape(1, NCLS),
    ]
    out = pl.pallas_call(
        _head_kernel,
        in_specs=[_bcast(op.shape) for op in head_ops],
        out_specs=_bcast((B, NCLS)),
        out_shape=jax.ShapeDtypeStruct((B, NCLS), f32),
    )(*head_ops)
    return out
